# baseline (speedup 1.0000x reference)
import functools
import math

import jax
import jax.numpy as jnp
from jax import lax
from jax.experimental import pallas as pl
from jax.experimental.pallas import tpu as pltpu
from jax.experimental.pallas import tpu_sc as plsc

_NC = 2
_NS = 16
_NW = _NC * _NS


def _mm_block(x_ref, w_ref, o_ref):
    o_ref[...] = jnp.dot(x_ref[...], w_ref[...], preferred_element_type=jnp.float32)


def _matmul(x, w, block_m):
    m, k = x.shape
    _, n = w.shape
    return pl.pallas_call(
        _mm_block,
        grid=(m // block_m,),
        in_specs=[
            pl.BlockSpec((block_m, k), lambda i: (i, 0)),
            pl.BlockSpec((k, n), lambda i: (0, 0)),
        ],
        out_specs=pl.BlockSpec((block_m, n), lambda i: (i, 0)),
        out_shape=jax.ShapeDtypeStruct((m, n), jnp.float32),
    )(x, w)


def _addmm_block(p_ref, w_ref, o_ref):
    h = p_ref[0] + p_ref[1]
    o_ref[...] = jnp.dot(h, w_ref[...], preferred_element_type=jnp.float32)


def _add_matmul(p, w, block_m, m):
    _, _, k = p.shape
    _, n = w.shape
    return pl.pallas_call(
        _addmm_block,
        grid=(m // block_m,),
        in_specs=[
            pl.BlockSpec((2, block_m, k), lambda i: (0, i, 0)),
            pl.BlockSpec((k, n), lambda i: (0, 0)),
        ],
        out_specs=pl.BlockSpec((block_m, n), lambda i: (i, 0)),
        out_shape=jax.ShapeDtypeStruct((m, n), jnp.float32),
    )(p, w)


def _add2(p):
    _, m, n = p.shape

    def body(p_ref, o_ref):
        o_ref[...] = p_ref[0] + p_ref[1]

    return pl.pallas_call(
        body,
        in_specs=[pl.BlockSpec((2, m, n), lambda: (0, 0, 0))],
        out_specs=pl.BlockSpec((m, n), lambda: (0, 0)),
        out_shape=jax.ShapeDtypeStruct((m, n), jnp.float32),
    )(p)


def _sc_gather_scatter(h, packed, chunk, nbuf, rpt):
    e = packed.shape[0]
    d = h.shape[1]
    ew = e // _NW
    nch = ew // chunk
    npad = rpt * _NS

    mesh = plsc.VectorSubcoreMesh(
        core_axis_name="c", subcore_axis_name="s", num_cores=_NC, num_subcores=_NS
    )

    @functools.partial(
        pl.kernel,
        out_type=jax.ShapeDtypeStruct((_NC, npad, d), jnp.float32),
        mesh=mesh,
        scratch_types=[
            pltpu.VMEM((nch, chunk), jnp.int32),
            [pltpu.VMEM((chunk,), jnp.int32) for _ in range(nbuf)],
            [pltpu.VMEM((chunk,), jnp.int32) for _ in range(nbuf)],
            [pltpu.VMEM((chunk, d), jnp.float32) for _ in range(nbuf)],
            pltpu.VMEM_SHARED((npad, d), jnp.float32),
            [pltpu.SemaphoreType.DMA for _ in range(nbuf)],
        ],
        compiler_params=pltpu.CompilerParams(use_tc_tiling_on_sc=False),
    )
    def edge_pass(h_hbm, pidx_hbm, z_hbm, out_hbm,
                  pidx, sidxs, didxs, rows, acc, sems):
        cid = lax.axis_index("c")
        sid = lax.axis_index("s")
        wid = sid * _NC + cid
        stripe = pl.ds(sid * rpt, rpt)
        pltpu.sync_copy(pidx_hbm.at[wid], pidx)
        pltpu.sync_copy(z_hbm, acc.at[stripe])
        plsc.subcore_barrier()

        def issue(k, b):
            for t in range(chunk // 16):
                v = pidx[k, pl.ds(t * 16, 16)]
                sidxs[b][pl.ds(t * 16, 16)] = v & 0xFFFF
                didxs[b][pl.ds(t * 16, 16)] = lax.shift_right_logical(v, 16)
            pltpu.async_copy(h_hbm.at[sidxs[b]], rows[b], sems[b])

        for b in range(nbuf - 1):
            issue(b, b)

        def step(j, carry):
            nxt = j + (nbuf - 1)
            for b in range(nbuf):
                @pl.when(jnp.logical_and(nxt < nch, nxt % nbuf == b))
                def _(b=b):
                    issue(nxt, b)

            for b in range(nbuf):
                @pl.when(j % nbuf == b)
                def _(b=b):
                    pltpu.make_async_copy(h_hbm.at[sidxs[b]], rows[b], sems[b]).wait()
                    pltpu.sync_copy(rows[b], acc.at[didxs[b]], add=True)

            return carry

        lax.fori_loop(0, nch, step, 0)
        plsc.subcore_barrier()
        pltpu.sync_copy(acc.at[stripe], out_hbm.at[cid, stripe])

    zeros = jnp.zeros((rpt, d), jnp.float32)
    pidx_r = packed.reshape(_NW, nch, chunk)
    return edge_pass(h, pidx_r, zeros)


def _sc_linearize(edge_index, n, spare, epad):
    e = edge_index.shape[1]
    ew = e // _NW
    ewp = epad // _NW
    npadw = ewp - ew
    alen = (127 + ew) // 128 * 128

    mesh = plsc.VectorSubcoreMesh(
        core_axis_name="c", subcore_axis_name="s", num_cores=_NC, num_subcores=_NS
    )

    @functools.partial(
        pl.kernel,
        out_type=jax.ShapeDtypeStruct((epad,), jnp.int32),
        mesh=mesh,
        scratch_types=[
            pltpu.VMEM((2, alen + 256), jnp.int32),
            pltpu.VMEM((ewp,), jnp.int32),
        ],
        compiler_params=pltpu.CompilerParams(use_tc_tiling_on_sc=True),
    )
    def linearize(ei_hbm, out_hbm, ebuf, pbuf):
        cid = lax.axis_index("c")
        sid = lax.axis_index("s")
        wid = sid * _NC + cid
        lead = (wid * ew) % 128
        abase = pl.multiple_of(wid * ew - lead, 128)
        pltpu.sync_copy(ei_hbm.at[:, pl.ds(abase, alen)], ebuf.at[:, pl.ds(0, alen)])

        def pack(t, carry):
            off = lead + t * 16
            p = ebuf[0, pl.ds(off, 16)] | (ebuf[1, pl.ds(off, 16)] << 16)
            pbuf[pl.ds(t * 16, 16)] = p
            return carry

        lax.fori_loop(0, ewp // 16, pack, 0)
        lanes = lax.iota(jnp.int32, 16)
        for t in range(npadw // 16):
            sp = wid * 16 + lanes
            dp = n + (wid * 16 + lanes) % spare
            pbuf[pl.ds(ew + t * 16, 16)] = sp | (dp << 16)
        pltpu.sync_copy(pbuf, out_hbm.at[pl.ds(wid * ewp, ewp)])

    return linearize(edge_index)


def kernel(x, edge_index, W1, W2):
    n = x.shape[0]
    d2 = W2.shape[1]

    e = edge_index.shape[1]
    epad = -(-e // (_NW * 128 * 8)) * (_NW * 128 * 8)
    rpt1 = -(-n // (_NS * 8)) * 8
    spare = rpt1 * _NS - n
    ei32 = edge_index.astype(jnp.int32)
    packed = _sc_linearize(ei32, n, spare, epad)

    g = 128 // math.gcd(d2, 128)
    rpt2 = -(-n // (_NS * 8 * g)) * 8 * g

    h1p = _matmul(x, W1, block_m=2000)
    p1 = _sc_gather_scatter(h1p, packed, 32, 8, rpt1)
    h2p = _add_matmul(p1, W2, block_m=2000, m=n)
    p2 = _sc_gather_scatter(h2p, packed, 128, 6, rpt2)
    npad2 = rpt2 * _NS
    p2r = p2.reshape(2, npad2 * d2 // 128, 128)
    s = _add2(p2r)
    return s.reshape(npad2, d2)[:n]


# speedup vs baseline: 1.0321x; 1.0321x over previous
import functools
import math

import jax
import jax.numpy as jnp
from jax import lax
from jax.experimental import pallas as pl
from jax.experimental.pallas import tpu as pltpu
from jax.experimental.pallas import tpu_sc as plsc

_NC = 2
_NS = 16
_NW = _NC * _NS


def _mm_block(x_ref, w_ref, o_ref):
    o_ref[...] = jnp.dot(x_ref[...], w_ref[...], preferred_element_type=jnp.float32)


def _matmul(x, w, block_m):
    m, k = x.shape
    _, n = w.shape
    return pl.pallas_call(
        _mm_block,
        grid=(m // block_m,),
        in_specs=[
            pl.BlockSpec((block_m, k), lambda i: (i, 0)),
            pl.BlockSpec((k, n), lambda i: (0, 0)),
        ],
        out_specs=pl.BlockSpec((block_m, n), lambda i: (i, 0)),
        out_shape=jax.ShapeDtypeStruct((m, n), jnp.float32),
    )(x, w)


def _addmm_block(p_ref, w_ref, o_ref):
    h = p_ref[0] + p_ref[1]
    o_ref[...] = jnp.dot(h, w_ref[...], preferred_element_type=jnp.float32)


def _add_matmul(p, w, block_m, m):
    _, _, k = p.shape
    _, n = w.shape
    return pl.pallas_call(
        _addmm_block,
        grid=(m // block_m,),
        in_specs=[
            pl.BlockSpec((2, block_m, k), lambda i: (0, i, 0)),
            pl.BlockSpec((k, n), lambda i: (0, 0)),
        ],
        out_specs=pl.BlockSpec((block_m, n), lambda i: (i, 0)),
        out_shape=jax.ShapeDtypeStruct((m, n), jnp.float32),
    )(p, w)


def _add2(p):
    _, m, n = p.shape

    def body(p_ref, o_ref):
        o_ref[...] = p_ref[0] + p_ref[1]

    return pl.pallas_call(
        body,
        in_specs=[pl.BlockSpec((2, m, n), lambda: (0, 0, 0))],
        out_specs=pl.BlockSpec((m, n), lambda: (0, 0)),
        out_shape=jax.ShapeDtypeStruct((m, n), jnp.float32),
    )(p)


def _sc_gather_scatter(h, packed, chunk, nbuf, rpt):
    e = packed.shape[0]
    d = h.shape[1]
    ew = e // _NW
    nch = ew // chunk
    npad = rpt * _NS

    mesh = plsc.VectorSubcoreMesh(
        core_axis_name="c", subcore_axis_name="s", num_cores=_NC, num_subcores=_NS
    )

    @functools.partial(
        pl.kernel,
        out_type=jax.ShapeDtypeStruct((_NC, npad, d), jnp.float32),
        mesh=mesh,
        scratch_types=[
            pltpu.VMEM((nch, chunk), jnp.int32),
            [pltpu.VMEM((chunk,), jnp.int32) for _ in range(nbuf)],
            [pltpu.VMEM((chunk,), jnp.int32) for _ in range(nbuf)],
            [pltpu.VMEM((chunk, d), jnp.float32) for _ in range(nbuf)],
            pltpu.VMEM_SHARED((npad, d), jnp.float32),
            [pltpu.SemaphoreType.DMA for _ in range(nbuf)],
        ],
        compiler_params=pltpu.CompilerParams(use_tc_tiling_on_sc=False),
    )
    def edge_pass(h_hbm, pidx_hbm, z_hbm, out_hbm,
                  pidx, sidxs, didxs, rows, acc, sems):
        cid = lax.axis_index("c")
        sid = lax.axis_index("s")
        wid = sid * _NC + cid
        stripe = pl.ds(sid * rpt, rpt)
        pltpu.sync_copy(pidx_hbm.at[wid], pidx)

        def issue(k, b):
            for t in range(chunk // 16):
                v = pidx[k, pl.ds(t * 16, 16)]
                sidxs[b][pl.ds(t * 16, 16)] = v & 0xFFFF
                didxs[b][pl.ds(t * 16, 16)] = lax.shift_right_logical(v, 16)
            pltpu.async_copy(h_hbm.at[sidxs[b]], rows[b], sems[b])

        for b in range(nbuf - 1):
            issue(b, b)
        pltpu.sync_copy(z_hbm, acc.at[stripe])
        plsc.subcore_barrier()

        def step(j, carry):
            nxt = j + (nbuf - 1)
            for b in range(nbuf):
                @pl.when(jnp.logical_and(nxt < nch, nxt % nbuf == b))
                def _(b=b):
                    issue(nxt, b)

            for b in range(nbuf):
                @pl.when(j % nbuf == b)
                def _(b=b):
                    pltpu.make_async_copy(h_hbm.at[sidxs[b]], rows[b], sems[b]).wait()
                    pltpu.sync_copy(rows[b], acc.at[didxs[b]], add=True)

            return carry

        lax.fori_loop(0, nch, step, 0)
        plsc.subcore_barrier()
        pltpu.sync_copy(acc.at[stripe], out_hbm.at[cid, stripe])

    zeros = jnp.zeros((rpt, d), jnp.float32)
    pidx_r = packed.reshape(_NW, nch, chunk)
    return edge_pass(h, pidx_r, zeros)


def _sc_linearize(edge_index, n, spare, epad):
    e = edge_index.shape[1]
    ew = e // _NW
    ewp = epad // _NW
    npadw = ewp - ew
    alen = (127 + ew) // 128 * 128

    mesh = plsc.VectorSubcoreMesh(
        core_axis_name="c", subcore_axis_name="s", num_cores=_NC, num_subcores=_NS
    )

    @functools.partial(
        pl.kernel,
        out_type=jax.ShapeDtypeStruct((epad,), jnp.int32),
        mesh=mesh,
        scratch_types=[
            pltpu.VMEM((2, alen + 256), jnp.int32),
            pltpu.VMEM((ewp,), jnp.int32),
        ],
        compiler_params=pltpu.CompilerParams(use_tc_tiling_on_sc=True),
    )
    def linearize(ei_hbm, out_hbm, ebuf, pbuf):
        cid = lax.axis_index("c")
        sid = lax.axis_index("s")
        wid = sid * _NC + cid
        lead = (wid * ew) % 128
        abase = pl.multiple_of(wid * ew - lead, 128)
        pltpu.sync_copy(ei_hbm.at[:, pl.ds(abase, alen)], ebuf.at[:, pl.ds(0, alen)])

        def pack(t, carry):
            off = lead + t * 16
            p = ebuf[0, pl.ds(off, 16)] | (ebuf[1, pl.ds(off, 16)] << 16)
            pbuf[pl.ds(t * 16, 16)] = p
            return carry

        lax.fori_loop(0, ewp // 16, pack, 0)
        lanes = lax.iota(jnp.int32, 16)
        for t in range(npadw // 16):
            sp = wid * 16 + lanes
            dp = n + (wid * 16 + lanes) % spare
            pbuf[pl.ds(ew + t * 16, 16)] = sp | (dp << 16)
        pltpu.sync_copy(pbuf, out_hbm.at[pl.ds(wid * ewp, ewp)])

    return linearize(edge_index)


def kernel(x, edge_index, W1, W2):
    n = x.shape[0]
    d2 = W2.shape[1]

    e = edge_index.shape[1]
    epad = -(-e // (_NW * 128 * 8)) * (_NW * 128 * 8)
    rpt1 = -(-n // (_NS * 8)) * 8
    spare = rpt1 * _NS - n
    ei32 = edge_index.astype(jnp.int32)
    packed = _sc_linearize(ei32, n, spare, epad)

    g = 128 // math.gcd(d2, 128)
    rpt2 = -(-n // (_NS * 8 * g)) * 8 * g

    h1p = _matmul(x, W1, block_m=2000)
    p1 = _sc_gather_scatter(h1p, packed, 64, 4, rpt1)
    h2p = _add_matmul(p1, W2, block_m=2000, m=n)
    p2 = _sc_gather_scatter(h2p, packed, 128, 8, rpt2)
    npad2 = rpt2 * _NS
    p2r = p2.reshape(2, npad2 * d2 // 128, 128)
    s = _add2(p2r)
    return s.reshape(npad2, d2)[:n]


# speedup vs baseline: 1.0355x; 1.0033x over previous
import functools
import math

import jax
import jax.numpy as jnp
from jax import lax
from jax.experimental import pallas as pl
from jax.experimental.pallas import tpu as pltpu
from jax.experimental.pallas import tpu_sc as plsc

_NC = 2
_NS = 16
_NW = _NC * _NS


def _mm_block(x_ref, w_ref, o_ref):
    o_ref[...] = jnp.dot(x_ref[...], w_ref[...], preferred_element_type=jnp.float32)


def _matmul(x, w, block_m):
    m, k = x.shape
    _, n = w.shape
    return pl.pallas_call(
        _mm_block,
        grid=(m // block_m,),
        in_specs=[
            pl.BlockSpec((block_m, k), lambda i: (i, 0)),
            pl.BlockSpec((k, n), lambda i: (0, 0)),
        ],
        out_specs=pl.BlockSpec((block_m, n), lambda i: (i, 0)),
        out_shape=jax.ShapeDtypeStruct((m, n), jnp.float32),
    )(x, w)


def _addmm_block(p_ref, w_ref, o_ref):
    h = p_ref[0] + p_ref[1]
    o_ref[...] = jnp.dot(h, w_ref[...], preferred_element_type=jnp.float32)


def _add_matmul(p, w, block_m, m):
    _, _, k = p.shape
    _, n = w.shape
    return pl.pallas_call(
        _addmm_block,
        grid=(m // block_m,),
        in_specs=[
            pl.BlockSpec((2, block_m, k), lambda i: (0, i, 0)),
            pl.BlockSpec((k, n), lambda i: (0, 0)),
        ],
        out_specs=pl.BlockSpec((block_m, n), lambda i: (i, 0)),
        out_shape=jax.ShapeDtypeStruct((m, n), jnp.float32),
    )(p, w)


def _add2(p):
    _, m, n = p.shape

    def body(p_ref, o_ref):
        o_ref[...] = p_ref[0] + p_ref[1]

    return pl.pallas_call(
        body,
        in_specs=[pl.BlockSpec((2, m, n), lambda: (0, 0, 0))],
        out_specs=pl.BlockSpec((m, n), lambda: (0, 0)),
        out_shape=jax.ShapeDtypeStruct((m, n), jnp.float32),
    )(p)


def _sc_gather_scatter(h, packed, chunk, nbuf, rpt):
    e = packed.shape[0]
    d = h.shape[1]
    ew = e // _NW
    nch = ew // chunk
    npad = rpt * _NS

    mesh = plsc.VectorSubcoreMesh(
        core_axis_name="c", subcore_axis_name="s", num_cores=_NC, num_subcores=_NS
    )

    @functools.partial(
        pl.kernel,
        out_type=jax.ShapeDtypeStruct((_NC, npad, d), jnp.float32),
        mesh=mesh,
        scratch_types=[
            pltpu.VMEM((nch, chunk), jnp.int32),
            [pltpu.VMEM((chunk,), jnp.int32) for _ in range(nbuf)],
            [pltpu.VMEM((chunk,), jnp.int32) for _ in range(nbuf)],
            [pltpu.VMEM((chunk, d), jnp.float32) for _ in range(nbuf)],
            pltpu.VMEM_SHARED((npad, d), jnp.float32),
            [pltpu.SemaphoreType.DMA for _ in range(nbuf)],
        ],
        compiler_params=pltpu.CompilerParams(use_tc_tiling_on_sc=False),
    )
    def edge_pass(h_hbm, pidx_hbm, z_hbm, out_hbm,
                  pidx, sidxs, didxs, rows, acc, sems):
        cid = lax.axis_index("c")
        sid = lax.axis_index("s")
        wid = sid * _NC + cid
        stripe = pl.ds(sid * rpt, rpt)
        pltpu.sync_copy(pidx_hbm.at[wid], pidx)

        def issue(k, b):
            for t in range(chunk // 16):
                v = pidx[k, pl.ds(t * 16, 16)]
                sidxs[b][pl.ds(t * 16, 16)] = v & 0xFFFF
                didxs[b][pl.ds(t * 16, 16)] = lax.shift_right_logical(v, 16)
            pltpu.async_copy(h_hbm.at[sidxs[b]], rows[b], sems[b])

        for b in range(nbuf - 1):
            issue(b, b)
        pltpu.sync_copy(z_hbm, acc.at[stripe])
        plsc.subcore_barrier()

        def step(j, carry):
            nxt = j + (nbuf - 1)
            for b in range(nbuf):
                @pl.when(jnp.logical_and(nxt < nch, nxt % nbuf == b))
                def _(b=b):
                    issue(nxt, b)

            for b in range(nbuf):
                @pl.when(j % nbuf == b)
                def _(b=b):
                    pltpu.make_async_copy(h_hbm.at[sidxs[b]], rows[b], sems[b]).wait()
                    pltpu.sync_copy(rows[b], acc.at[didxs[b]], add=True)

            return carry

        lax.fori_loop(0, nch, step, 0)
        plsc.subcore_barrier()
        pltpu.sync_copy(acc.at[stripe], out_hbm.at[cid, stripe])

    zeros = jnp.zeros((rpt, d), jnp.float32)
    pidx_r = packed.reshape(_NW, nch, chunk)
    return edge_pass(h, pidx_r, zeros)


def _sc_linearize(edge_index, n, spare, epad):
    e = edge_index.shape[1]
    ew = e // _NW
    ewp = epad // _NW
    npadw = ewp - ew
    alen = (127 + ew) // 128 * 128

    mesh = plsc.VectorSubcoreMesh(
        core_axis_name="c", subcore_axis_name="s", num_cores=_NC, num_subcores=_NS
    )

    @functools.partial(
        pl.kernel,
        out_type=jax.ShapeDtypeStruct((epad,), jnp.int32),
        mesh=mesh,
        scratch_types=[
            pltpu.VMEM((2, alen + 256), jnp.int32),
            pltpu.VMEM((ewp,), jnp.int32),
        ],
        compiler_params=pltpu.CompilerParams(use_tc_tiling_on_sc=True),
    )
    def linearize(ei_hbm, out_hbm, ebuf, pbuf):
        cid = lax.axis_index("c")
        sid = lax.axis_index("s")
        wid = sid * _NC + cid
        lead = (wid * ew) % 128
        abase = pl.multiple_of(wid * ew - lead, 128)
        pltpu.sync_copy(ei_hbm.at[:, pl.ds(abase, alen)], ebuf.at[:, pl.ds(0, alen)])

        def pack(t, carry):
            off = lead + t * 16
            p = ebuf[0, pl.ds(off, 16)] | (ebuf[1, pl.ds(off, 16)] << 16)
            pbuf[pl.ds(t * 16, 16)] = p
            return carry

        lax.fori_loop(0, ewp // 16, pack, 0)
        lanes = lax.iota(jnp.int32, 16)
        for t in range(npadw // 16):
            sp = wid * 16 + lanes
            dp = n + (wid * 16 + lanes) % spare
            pbuf[pl.ds(ew + t * 16, 16)] = sp | (dp << 16)
        pltpu.sync_copy(pbuf, out_hbm.at[pl.ds(wid * ewp, ewp)])

    return linearize(edge_index)


def kernel(x, edge_index, W1, W2):
    n = x.shape[0]
    d2 = W2.shape[1]

    e = edge_index.shape[1]
    epad = -(-e // (_NW * 128 * 8)) * (_NW * 128 * 8)
    rpt1 = -(-n // (_NS * 8)) * 8
    spare = rpt1 * _NS - n
    ei32 = edge_index.astype(jnp.int32)
    packed = _sc_linearize(ei32, n, spare, epad)

    g = 128 // math.gcd(d2, 128)
    rpt2 = -(-n // (_NS * 8 * g)) * 8 * g

    h1p = _matmul(x, W1, block_m=10000)
    p1 = _sc_gather_scatter(h1p, packed, 64, 4, rpt1)
    h2p = _add_matmul(p1, W2, block_m=10000, m=n)
    p2 = _sc_gather_scatter(h2p, packed, 128, 8, rpt2)
    npad2 = rpt2 * _NS
    p2r = p2.reshape(2, npad2 * d2 // 128, 128)
    s = _add2(p2r)
    return s.reshape(npad2, d2)[:n]
